# baseline (speedup 1.0000x reference)
import jax
import jax.numpy as jnp
from jax import lax
from jax.experimental import pallas as pl
from jax.experimental.pallas import tpu as pltpu
from jax.experimental.pallas import tpu_sc as plsc

_B, _C, _H, _W = 4, 64, 256, 512
_NW = 32
_ROWS = _B * _H
_RPW = _ROWS // _NW
_NG = _W // 16
_CH = _C // 2


_UNROLL = 2


def _warp_stream(in_v, out_v, units, depth=10):
    tasks = [(u, c) for u in range(len(units)) for c in range(_CH)]
    pend = []
    nxt = []
    zv = jnp.zeros((16,), jnp.int32)
    for i in range(len(tasks) + depth):
        if i < len(tasks):
            u, c = tasks[i]
            ia, ib = units[u][1][0], units[u][1][1]
            cv = zv + c
            pend.append((u, c, plsc.load_gather(in_v, [cv, ia]),
                         plsc.load_gather(in_v, [cv, ib])))
        for u in range(len(units)):
            if i == 16 + u * _CH:
                nxt.append(units[u][2]())
        if len(pend) > depth or (i >= len(tasks) and pend):
            uu, cc, ga, gb = pend.pop(0)
            col = units[uu][0]
            wa, wb = units[uu][1][2], units[uu][1][3]
            out_v[cc, pl.ds(col, 16)] = wa * ga + wb * gb
    return tuple(nxt)


def _warp_body(right_hbm, disp_hbm, out_hbm,
               in0a_v, in0b_v, in1a_v, in1b_v, out0_v, out1_v,
               dispa_v, dispb_v, disp1_v, oi0_v, oi1_v,
               ii0a_v, ii0b_v, ii1a_v, ii1b_v,
               sem_i0a, sem_i0b, sem_i1a, sem_i1b, sem_o0, sem_o1):
    cid = lax.axis_index("c")
    sid = lax.axis_index("s")
    wid = sid * 2 + cid

    def chan_base(row, half):
        b = row // _H
        h = row - b * _H
        return b * (_C * _H) + h + half * (_CH * _H)

    def fill_rowidx(row, half, iv):
        base = chan_base(row, half)
        iv[pl.ds(0, 16)] = base + lax.iota(jnp.int32, 16) * _H
        iv[pl.ds(16, 16)] = base + (lax.iota(jnp.int32, 16) + 16) * _H

    def issue_in(row, half, buf, iiv, sem):
        fill_rowidx(row, half, iiv)
        pltpu.async_copy(right_hbm.at[iiv], buf, sem)

    def wait_in(buf, iiv, sem):
        pltpu.make_async_copy(right_hbm.at[iiv], buf, sem).wait()

    fill_outidx = fill_rowidx

    def make_weights(col, dsp, keep=None):
        d = dsp[pl.ds(col, 16)]
        if keep is not None:
            keep[pl.ds(col, 16)] = d
        wvec = lax.iota(jnp.int32, 16) + col
        ry = d + wvec.astype(jnp.float32)
        t = ry.astype(jnp.int32)
        tf = t.astype(jnp.float32)
        adj = tf > ry
        fi = t - jnp.where(adj, 1, 0)
        fa = tf - jnp.where(adj, 1.0, 0.0)
        wb = ry - fa
        wa = 1.0 - wb
        valid = (ry >= 0.0) & (ry <= float(_W - 1))
        wa = jnp.where(valid, wa, 0.0)
        wb = jnp.where(valid, wb, 0.0)
        ia = jnp.clip(fi, 0, _W - 1)
        ib = jnp.clip(fi + 1, 0, _W - 1)
        return ia, ib, wa, wb

    _STEP = 16 * _UNROLL

    def compute_half(in_v, out_v, get_w):
        def g(gi, w4s):
            base = gi * _STEP
            units = [(base + u * 16, w4s[u],
                      (lambda uu: lambda: get_w(base + _STEP + uu * 16))(u))
                     for u in range(_UNROLL)]
            return _warp_stream(in_v, out_v, units)

        w0 = tuple(get_w(u * 16) for u in range(_UNROLL))
        lax.fori_loop(0, _NG // _UNROLL, g, w0)

    def compute_h0(in_v, out_v, dspbuf):
        compute_half(in_v, out_v,
                     lambda col: make_weights(col, dspbuf, keep=disp1_v))

    def compute_h1(in_v, out_v):
        compute_half(in_v, out_v, lambda col: make_weights(col, disp1_v))

    row0 = wid * _RPW
    issue_in(row0, 0, in0a_v, ii0a_v, sem_i0a)
    pltpu.async_copy(disp_hbm.at[row0], dispa_v.at[pl.ds(0, _W)], sem_i0a)
    issue_in(row0, 1, in1a_v, ii1a_v, sem_i1a)
    issue_in(row0 + 1, 0, in0b_v, ii0b_v, sem_i0b)
    pltpu.async_copy(disp_hbm.at[row0 + 1], dispb_v.at[pl.ds(0, _W)], sem_i0b)
    issue_in(row0 + 1, 1, in1b_v, ii1b_v, sem_i1b)

    def row_phase(row, in0x, ii0x, sem0x, in1x, ii1x, sem1x, dispx):
        wait_in(in0x, ii0x, sem0x)
        pltpu.make_async_copy(disp_hbm.at[0], dispx.at[pl.ds(0, _W)],
                              sem0x).wait()

        @pl.when(row > row0)
        def _():
            pltpu.make_async_copy(out0_v, out_hbm.at[oi0_v], sem_o0).wait()

        compute_h0(in0x, out0_v, dispx)
        fill_outidx(row, 0, oi0_v)
        pltpu.async_copy(out0_v, out_hbm.at[oi0_v], sem_o0)

        @pl.when(row + 2 < row0 + _RPW)
        def _():
            issue_in(row + 2, 0, in0x, ii0x, sem0x)
            pltpu.async_copy(disp_hbm.at[row + 2], dispx.at[pl.ds(0, _W)],
                             sem0x)

        wait_in(in1x, ii1x, sem1x)

        @pl.when(row > row0)
        def _():
            pltpu.make_async_copy(out1_v, out_hbm.at[oi1_v], sem_o1).wait()

        compute_h1(in1x, out1_v)
        fill_outidx(row, 1, oi1_v)
        pltpu.async_copy(out1_v, out_hbm.at[oi1_v], sem_o1)

        @pl.when(row + 2 < row0 + _RPW)
        def _():
            issue_in(row + 2, 1, in1x, ii1x, sem1x)

    def pair_body(m, carry):
        row = row0 + 2 * m
        row_phase(row, in0a_v, ii0a_v, sem_i0a, in1a_v, ii1a_v, sem_i1a,
                  dispa_v)
        row_phase(row + 1, in0b_v, ii0b_v, sem_i0b, in1b_v, ii1b_v,
                  sem_i1b, dispb_v)
        return carry

    lax.fori_loop(0, _RPW // 2, pair_body, 0)
    pltpu.make_async_copy(out0_v, out_hbm.at[oi0_v], sem_o0).wait()
    pltpu.make_async_copy(out1_v, out_hbm.at[oi1_v], sem_o1).wait()


def kernel(right_input, disparity_samples):
    right_r = right_input.reshape(_B * _C * _H, _W)
    disp_r = disparity_samples.reshape(_B * _H, _W)
    mesh = plsc.VectorSubcoreMesh(core_axis_name="c", subcore_axis_name="s")
    out = pl.kernel(
        _warp_body,
        mesh=mesh,
        compiler_params=pltpu.CompilerParams(needs_layout_passes=False),
        out_type=jax.ShapeDtypeStruct((_B * _C * _H, _W), jnp.float32),
        scratch_types=[
            pltpu.VMEM((_CH, _W), jnp.float32),
            pltpu.VMEM((_CH, _W), jnp.float32),
            pltpu.VMEM((_CH, _W), jnp.float32),
            pltpu.VMEM((_CH, _W), jnp.float32),
            pltpu.VMEM((_CH, _W), jnp.float32),
            pltpu.VMEM((_CH, _W), jnp.float32),
            pltpu.VMEM((_W + 2 * 16 * _UNROLL,), jnp.float32),
            pltpu.VMEM((_W + 2 * 16 * _UNROLL,), jnp.float32),
            pltpu.VMEM((_W + 2 * 16 * _UNROLL,), jnp.float32),
            pltpu.VMEM((_CH,), jnp.int32),
            pltpu.VMEM((_CH,), jnp.int32),
            pltpu.VMEM((_CH,), jnp.int32),
            pltpu.VMEM((_CH,), jnp.int32),
            pltpu.VMEM((_CH,), jnp.int32),
            pltpu.VMEM((_CH,), jnp.int32),
            pltpu.SemaphoreType.DMA,
            pltpu.SemaphoreType.DMA,
            pltpu.SemaphoreType.DMA,
            pltpu.SemaphoreType.DMA,
            pltpu.SemaphoreType.DMA,
            pltpu.SemaphoreType.DMA,
        ],
    )(right_r, disp_r)
    return out.reshape(_B, _C, _H, _W)


# speedup vs baseline: 1.0765x; 1.0765x over previous
import jax
import jax.numpy as jnp
from jax import lax
from jax.experimental import pallas as pl
from jax.experimental.pallas import tpu as pltpu
from jax.experimental.pallas import tpu_sc as plsc

_B, _C, _H, _W = 4, 64, 256, 512
_NW = 32
_ROWS = _B * _H
_RPW = _ROWS // _NW
_NG = _W // 16
_CH = _C // 2


_UNROLL = 2


def _warp_stream(in_v, out_v, units, depth=6):
    tasks = [(u, c) for u in range(len(units)) for c in range(_CH)]
    pend = []
    nxt = []
    zv = jnp.zeros((16,), jnp.int32)
    for i in range(len(tasks) + depth):
        if i < len(tasks):
            u, c = tasks[i]
            ia, ib = units[u][1][0], units[u][1][1]
            cv = zv + c
            pend.append((u, c, plsc.load_gather(in_v, [cv, ia]),
                         plsc.load_gather(in_v, [cv, ib])))
        for u in range(len(units)):
            if i == 16 + u * _CH:
                nxt.append(units[u][2]())
        if len(pend) > depth or (i >= len(tasks) and pend):
            uu, cc, ga, gb = pend.pop(0)
            col = units[uu][0]
            wa, wb = units[uu][1][2], units[uu][1][3]
            out_v[cc, pl.ds(col, 16)] = wa * ga + wb * gb
    return tuple(nxt)


def _warp_body(right_hbm, disp_hbm, out_hbm,
               in0a_v, in0b_v, in1a_v, in1b_v, out0_v, out1_v,
               dispa_v, dispb_v, disp1_v, oi0_v, oi1_v,
               ii0a_v, ii0b_v, ii1a_v, ii1b_v,
               sem_i0a, sem_i0b, sem_i1a, sem_i1b, sem_o0, sem_o1):
    cid = lax.axis_index("c")
    sid = lax.axis_index("s")
    wid = sid * 2 + cid

    def chan_base(row, half):
        b = row // _H
        h = row - b * _H
        return b * (_C * _H) + h + half * (_CH * _H)

    def fill_rowidx(row, half, iv):
        base = chan_base(row, half)
        iv[pl.ds(0, 16)] = base + lax.iota(jnp.int32, 16) * _H
        iv[pl.ds(16, 16)] = base + (lax.iota(jnp.int32, 16) + 16) * _H

    def issue_in(row, half, buf, iiv, sem):
        fill_rowidx(row, half, iiv)
        pltpu.async_copy(right_hbm.at[iiv], buf, sem)

    def wait_in(buf, iiv, sem):
        pltpu.make_async_copy(right_hbm.at[iiv], buf, sem).wait()

    fill_outidx = fill_rowidx

    def make_weights(col, dsp, keep=None):
        d = dsp[pl.ds(col, 16)]
        if keep is not None:
            keep[pl.ds(col, 16)] = d
        wvec = lax.iota(jnp.int32, 16) + col
        ry = d + wvec.astype(jnp.float32)
        t = ry.astype(jnp.int32)
        tf = t.astype(jnp.float32)
        adj = tf > ry
        fi = t - jnp.where(adj, 1, 0)
        fa = tf - jnp.where(adj, 1.0, 0.0)
        wb = ry - fa
        wa = 1.0 - wb
        valid = (ry >= 0.0) & (ry <= float(_W - 1))
        wa = jnp.where(valid, wa, 0.0)
        wb = jnp.where(valid, wb, 0.0)
        ia = jnp.clip(fi, 0, _W - 1)
        ib = jnp.clip(fi + 1, 0, _W - 1)
        return ia, ib, wa, wb

    _STEP = 16 * _UNROLL

    def compute_half(in_v, out_v, get_w):
        def g(gi, w4s):
            base = gi * _STEP
            units = [(base + u * 16, w4s[u],
                      (lambda uu: lambda: get_w(base + _STEP + uu * 16))(u))
                     for u in range(_UNROLL)]
            return _warp_stream(in_v, out_v, units)

        w0 = tuple(get_w(u * 16) for u in range(_UNROLL))
        lax.fori_loop(0, _NG // _UNROLL, g, w0)

    def compute_h0(in_v, out_v, dspbuf):
        compute_half(in_v, out_v,
                     lambda col: make_weights(col, dspbuf, keep=disp1_v))

    def compute_h1(in_v, out_v):
        compute_half(in_v, out_v, lambda col: make_weights(col, disp1_v))

    row0 = wid * _RPW
    issue_in(row0, 0, in0a_v, ii0a_v, sem_i0a)
    pltpu.async_copy(disp_hbm.at[row0], dispa_v.at[pl.ds(0, _W)], sem_i0a)
    issue_in(row0, 1, in1a_v, ii1a_v, sem_i1a)
    issue_in(row0 + 1, 0, in0b_v, ii0b_v, sem_i0b)
    pltpu.async_copy(disp_hbm.at[row0 + 1], dispb_v.at[pl.ds(0, _W)], sem_i0b)
    issue_in(row0 + 1, 1, in1b_v, ii1b_v, sem_i1b)

    def row_phase(row, in0x, ii0x, sem0x, in1x, ii1x, sem1x, dispx):
        wait_in(in0x, ii0x, sem0x)
        pltpu.make_async_copy(disp_hbm.at[0], dispx.at[pl.ds(0, _W)],
                              sem0x).wait()

        @pl.when(row > row0)
        def _():
            pltpu.make_async_copy(out0_v, out_hbm.at[oi0_v], sem_o0).wait()

        compute_h0(in0x, out0_v, dispx)
        fill_outidx(row, 0, oi0_v)
        pltpu.async_copy(out0_v, out_hbm.at[oi0_v], sem_o0)

        @pl.when(row + 2 < row0 + _RPW)
        def _():
            issue_in(row + 2, 0, in0x, ii0x, sem0x)
            pltpu.async_copy(disp_hbm.at[row + 2], dispx.at[pl.ds(0, _W)],
                             sem0x)

        wait_in(in1x, ii1x, sem1x)

        @pl.when(row > row0)
        def _():
            pltpu.make_async_copy(out1_v, out_hbm.at[oi1_v], sem_o1).wait()

        compute_h1(in1x, out1_v)
        fill_outidx(row, 1, oi1_v)
        pltpu.async_copy(out1_v, out_hbm.at[oi1_v], sem_o1)

        @pl.when(row + 2 < row0 + _RPW)
        def _():
            issue_in(row + 2, 1, in1x, ii1x, sem1x)

    def pair_body(m, carry):
        row = row0 + 2 * m
        row_phase(row, in0a_v, ii0a_v, sem_i0a, in1a_v, ii1a_v, sem_i1a,
                  dispa_v)
        row_phase(row + 1, in0b_v, ii0b_v, sem_i0b, in1b_v, ii1b_v,
                  sem_i1b, dispb_v)
        return carry

    lax.fori_loop(0, _RPW // 2, pair_body, 0)
    pltpu.make_async_copy(out0_v, out_hbm.at[oi0_v], sem_o0).wait()
    pltpu.make_async_copy(out1_v, out_hbm.at[oi1_v], sem_o1).wait()


def kernel(right_input, disparity_samples):
    right_r = right_input.reshape(_B * _C * _H, _W)
    disp_r = disparity_samples.reshape(_B * _H, _W)
    mesh = plsc.VectorSubcoreMesh(core_axis_name="c", subcore_axis_name="s")
    out = pl.kernel(
        _warp_body,
        mesh=mesh,
        compiler_params=pltpu.CompilerParams(needs_layout_passes=False),
        out_type=jax.ShapeDtypeStruct((_B * _C * _H, _W), jnp.float32),
        scratch_types=[
            pltpu.VMEM((_CH, _W), jnp.float32),
            pltpu.VMEM((_CH, _W), jnp.float32),
            pltpu.VMEM((_CH, _W), jnp.float32),
            pltpu.VMEM((_CH, _W), jnp.float32),
            pltpu.VMEM((_CH, _W), jnp.float32),
            pltpu.VMEM((_CH, _W), jnp.float32),
            pltpu.VMEM((_W + 2 * 16 * _UNROLL,), jnp.float32),
            pltpu.VMEM((_W + 2 * 16 * _UNROLL,), jnp.float32),
            pltpu.VMEM((_W + 2 * 16 * _UNROLL,), jnp.float32),
            pltpu.VMEM((_CH,), jnp.int32),
            pltpu.VMEM((_CH,), jnp.int32),
            pltpu.VMEM((_CH,), jnp.int32),
            pltpu.VMEM((_CH,), jnp.int32),
            pltpu.VMEM((_CH,), jnp.int32),
            pltpu.VMEM((_CH,), jnp.int32),
            pltpu.SemaphoreType.DMA,
            pltpu.SemaphoreType.DMA,
            pltpu.SemaphoreType.DMA,
            pltpu.SemaphoreType.DMA,
            pltpu.SemaphoreType.DMA,
            pltpu.SemaphoreType.DMA,
        ],
    )(right_r, disp_r)
    return out.reshape(_B, _C, _H, _W)
